# baseline (speedup 1.0000x reference)
import functools

import jax
import jax.numpy as jnp
from jax import lax
from jax.experimental import pallas as pl
from jax.experimental.pallas import tpu as pltpu
from jax.experimental.pallas import tpu_sc as plsc

_VOCAB = 100000
_EMBED = 64
_B = 1024
_CTX = 10

_NC = 2
_NS = 16
_NW = _NC * _NS
_IDX_PER_W = _B * _CTX // _NW
_ROWS_PER_W = _B // _NW
_IDX_CHUNK = 80
_N_CHUNKS = _IDX_PER_W // _IDX_CHUNK


def _pool_sc(context, emb_table):
    idx = context.astype(jnp.int32).reshape(_B * _CTX // _IDX_CHUNK, _IDX_CHUNK)

    mesh = plsc.VectorSubcoreMesh(core_axis_name="c", subcore_axis_name="s")

    @functools.partial(
        pl.kernel,
        out_type=jax.ShapeDtypeStruct((_B, _EMBED), jnp.float32),
        mesh=mesh,
        scratch_types=[
            pltpu.VMEM((_N_CHUNKS, _IDX_CHUNK), jnp.int32),
            pltpu.VMEM((_IDX_PER_W, _EMBED), jnp.float32),
            pltpu.VMEM((_ROWS_PER_W, _EMBED), jnp.float32),
            pltpu.SemaphoreType.DMA,
        ],
        compiler_params=pltpu.CompilerParams(use_tc_tiling_on_sc=False),
    )
    def pool(idx_hbm, table_hbm, out_hbm, idx_v, rows_v, pooled_v, sem):
        wid = lax.axis_index("s") * _NC + lax.axis_index("c")
        pltpu.sync_copy(idx_hbm.at[pl.ds(wid * _N_CHUNKS, _N_CHUNKS)], idx_v)
        cps = [
            pltpu.async_copy(
                table_hbm.at[idx_v.at[c]],
                rows_v.at[pl.ds(c * _IDX_CHUNK, _IDX_CHUNK)],
                sem,
            )
            for c in range(_N_CHUNKS)
        ]
        for cp in cps:
            cp.wait()

        def row_body(r, carry):
            for v in range(_EMBED // 16):
                acc = rows_v[r * _CTX, pl.ds(v * 16, 16)]
                for j in range(1, _CTX):
                    acc = acc + rows_v[r * _CTX + j, pl.ds(v * 16, 16)]
                pooled_v[r, pl.ds(v * 16, 16)] = acc * (1.0 / _CTX)
            return carry

        lax.fori_loop(0, _ROWS_PER_W, row_body, 0)
        pltpu.sync_copy(pooled_v, out_hbm.at[pl.ds(wid * _ROWS_PER_W, _ROWS_PER_W)])

    return pool(idx, emb_table)


_BM = 32


def _tc_body(x_ref, w_ref, b_ref, o_ref):
    x = x_ref[...]
    logits = (
        jnp.dot(x, w_ref[...], preferred_element_type=jnp.float32) + b_ref[...]
    )
    m = jnp.max(logits, axis=1, keepdims=True)
    e = jnp.exp(logits - m)
    o_ref[...] = e * (1.0 / jnp.sum(e, axis=1, keepdims=True))


def _project_softmax(pooled, W, b):
    return pl.pallas_call(
        _tc_body,
        grid=(_B // _BM,),
        in_specs=[
            pl.BlockSpec((_BM, _EMBED), lambda i: (i, 0)),
            pl.BlockSpec((_EMBED, _VOCAB), lambda i: (0, 0)),
            pl.BlockSpec((1, _VOCAB), lambda i: (0, 0)),
        ],
        out_specs=pl.BlockSpec((_BM, _VOCAB), lambda i: (i, 0)),
        out_shape=jax.ShapeDtypeStruct((_B, _VOCAB), jnp.float32),
        compiler_params=pltpu.CompilerParams(vmem_limit_bytes=112 * 1024 * 1024),
    )(pooled, W, b.reshape(1, _VOCAB))


def kernel(context, emb_table, W, b):
    pooled = _pool_sc(context, emb_table)
    return _project_softmax(pooled, W, b)


# speedup vs baseline: 1.0561x; 1.0561x over previous
import functools

import jax
import jax.numpy as jnp
from jax import lax
from jax.experimental import pallas as pl
from jax.experimental.pallas import tpu as pltpu
from jax.experimental.pallas import tpu_sc as plsc

_VOCAB = 100000
_EMBED = 64
_B = 1024
_CTX = 10

_NC = 2
_NS = 16
_NW = _NC * _NS
_IDX_PER_W = _B * _CTX // _NW
_ROWS_PER_W = _B // _NW
_IDX_CHUNK = 80
_N_CHUNKS = _IDX_PER_W // _IDX_CHUNK


def _pool_sc(context, emb_table):
    idx = context.astype(jnp.int32).reshape(_B * _CTX // _IDX_CHUNK, _IDX_CHUNK)

    mesh = plsc.VectorSubcoreMesh(core_axis_name="c", subcore_axis_name="s")

    @functools.partial(
        pl.kernel,
        out_type=jax.ShapeDtypeStruct((_B, _EMBED), jnp.float32),
        mesh=mesh,
        scratch_types=[
            pltpu.VMEM((_N_CHUNKS, _IDX_CHUNK), jnp.int32),
            pltpu.VMEM((_IDX_PER_W, _EMBED), jnp.float32),
            pltpu.VMEM((_ROWS_PER_W, _EMBED), jnp.float32),
            pltpu.SemaphoreType.DMA,
        ],
        compiler_params=pltpu.CompilerParams(use_tc_tiling_on_sc=False),
    )
    def pool(idx_hbm, table_hbm, out_hbm, idx_v, rows_v, pooled_v, sem):
        wid = lax.axis_index("s") * _NC + lax.axis_index("c")
        pltpu.sync_copy(idx_hbm.at[pl.ds(wid * _N_CHUNKS, _N_CHUNKS)], idx_v)
        cps = [
            pltpu.async_copy(
                table_hbm.at[idx_v.at[c]],
                rows_v.at[pl.ds(c * _IDX_CHUNK, _IDX_CHUNK)],
                sem,
            )
            for c in range(_N_CHUNKS)
        ]
        for cp in cps:
            cp.wait()

        def row_body(r, carry):
            for v in range(_EMBED // 16):
                acc = rows_v[r * _CTX, pl.ds(v * 16, 16)]
                for j in range(1, _CTX):
                    acc = acc + rows_v[r * _CTX + j, pl.ds(v * 16, 16)]
                pooled_v[r, pl.ds(v * 16, 16)] = acc * (1.0 / _CTX)
            return carry

        lax.fori_loop(0, _ROWS_PER_W, row_body, 0)
        pltpu.sync_copy(pooled_v, out_hbm.at[pl.ds(wid * _ROWS_PER_W, _ROWS_PER_W)])

    return pool(idx, emb_table)


_BM = 32
_VC = 8192
_CHUNKS = [(s, min(_VC, _VOCAB - s)) for s in range(0, _VOCAB, _VC)]


def _tc_body(x_ref, w_hbm, b_ref, o_ref, w_vmem, sem):
    @pl.when(pl.program_id(0) == 0)
    def _():
        pltpu.make_async_copy(w_hbm, w_vmem, sem).start()
        pltpu.make_async_copy(w_hbm, w_vmem, sem).wait()

    x = x_ref[...]
    m = jnp.full((_BM, 1), -jnp.inf, jnp.float32)
    for st, sz in _CHUNKS:
        l = (
            jnp.dot(x, w_vmem[:, st : st + sz], preferred_element_type=jnp.float32)
            + b_ref[:, st : st + sz]
        )
        o_ref[:, st : st + sz] = l
        m = jnp.maximum(m, jnp.max(l, axis=1, keepdims=True))
    s = jnp.zeros((_BM, 1), jnp.float32)
    for st, sz in _CHUNKS:
        e = jnp.exp(o_ref[:, st : st + sz] - m)
        o_ref[:, st : st + sz] = e
        s = s + jnp.sum(e, axis=1, keepdims=True)
    r = 1.0 / s
    for st, sz in _CHUNKS:
        o_ref[:, st : st + sz] = o_ref[:, st : st + sz] * r


def _project_softmax(pooled, W, b):
    return pl.pallas_call(
        _tc_body,
        grid=(_B // _BM,),
        in_specs=[
            pl.BlockSpec((_BM, _EMBED), lambda i: (i, 0)),
            pl.BlockSpec(memory_space=pl.ANY),
            pl.BlockSpec((1, _VOCAB), lambda i: (0, 0)),
        ],
        out_specs=pl.BlockSpec((_BM, _VOCAB), lambda i: (i, 0)),
        out_shape=jax.ShapeDtypeStruct((_B, _VOCAB), jnp.float32),
        scratch_shapes=[
            pltpu.VMEM((_EMBED, _VOCAB), jnp.bfloat16),
            pltpu.SemaphoreType.DMA,
        ],
        compiler_params=pltpu.CompilerParams(
            dimension_semantics=("arbitrary",),
            vmem_limit_bytes=63 * 1024 * 1024,
        ),
    )(pooled.astype(jnp.bfloat16), W.astype(jnp.bfloat16), b.reshape(1, _VOCAB))


def kernel(context, emb_table, W, b):
    pooled = _pool_sc(context, emb_table)
    return _project_softmax(pooled, W, b)
